# baseline (speedup 1.0000x reference)
import dataclasses
import functools

import numpy as np
import jax
import jax.numpy as jnp
from jax import lax
from jax.experimental import pallas as pl
from jax.experimental.pallas import tpu as pltpu
from jax.experimental.pallas import tpu_sc as plsc

N = 10000
E = 160000
H = 256
HALF = 128

NC = 2
NS = 16
CHUNK = 64
NCHUNK_TOT = E // CHUNK
ORDS = NCHUNK_TOT // NS
LEFT = NCHUNK_TOT - ORDS * NS
MAIN_ITERS = ORDS // 4
EPI = ORDS - MAIN_ITERS * 4
ROWS_MAIN = 624
ROWS_TAIL = N - ROWS_MAIN * NS
ZROWS = 48

BN = 2000
NB = N // BN


def _mm_body(x_ref, w_ref, o_ref):
    o_ref[0] = lax.dot_general(
        x_ref[...], w_ref[0],
        (((1,), (0,)), ((), ())),
        preferred_element_type=jnp.float32,
        precision=lax.Precision.HIGHEST).astype(jnp.bfloat16)


def _matmul_h(x, w6t):
    return pl.pallas_call(
        _mm_body,
        grid=(NB, 6),
        in_specs=[
            pl.BlockSpec((BN, H), lambda b, g: (b, 0)),
            pl.BlockSpec((1, H, HALF), lambda b, g: (g, 0, 0)),
        ],
        out_specs=pl.BlockSpec((1, BN, HALF), lambda b, g: (g, b, 0)),
        out_shape=jax.ShapeDtypeStruct((6, N, HALF), jnp.bfloat16),
    )(x, w6t)


def _make_spmm():
    mesh = plsc.VectorSubcoreMesh(core_axis_name="c", subcore_axis_name="s")
    cp = pltpu.CompilerParams()
    if "needs_layout_passes" in pltpu.CompilerParams.__dataclass_fields__:
        cp = dataclasses.replace(cp, needs_layout_passes=False)
    if "use_tc_tiling_on_sc" in pltpu.CompilerParams.__dataclass_fields__:
        cp = dataclasses.replace(cp, use_tc_tiling_on_sc=False)

    @functools.partial(
        pl.kernel,
        mesh=mesh,
        compiler_params=cp,
        out_type=jax.ShapeDtypeStruct((6 * N, HALF), jnp.float32),
        scratch_types=(
            [pltpu.VMEM((CHUNK,), jnp.int32) for _ in range(4)]
            + [pltpu.VMEM((CHUNK,), jnp.int32) for _ in range(4)]
            + [pltpu.VMEM((CHUNK,), jnp.float32) for _ in range(4)]
            + [pltpu.VMEM((CHUNK,), jnp.int32) for _ in range(2)]
            + [pltpu.VMEM((CHUNK, HALF // 2), jnp.int32) for _ in range(2)]
            + [pltpu.VMEM((CHUNK, HALF), jnp.float32) for _ in range(2)]
            + [
                pltpu.VMEM((ZROWS, HALF), jnp.float32),
                pltpu.VMEM_SHARED((N, HALF), jnp.float32),
            ]
            + [pltpu.SemaphoreType.DMA for _ in range(9)]
        ),
    )
    def spmm(h_hbm, a0_hbm, w0_hbm, a1_hbm, w1_hbm, a2_hbm, w2_hbm, out_hbm,
             src0, src1, src2, src3, dst0, dst1, dst2, dst3,
             wv0, wv1, wv2, wv3, dsc0, dsc1, rbf0, rbf1, rows0, rows1,
             zero_v, acc_sh, ls0, ls1, ls2, ls3, gs0, gs1, as0, as1, zs):
        c = lax.axis_index("c")
        s = lax.axis_index("s")
        srcs = (src0, src1, src2, src3)
        dsts = (dst0, dst1, dst2, dst3)
        wvs = (wv0, wv1, wv2, wv3)
        dscs = (dsc0, dsc1)
        rbfs = (rbf0, rbf1)
        rows = (rows0, rows1)
        lsems = (ls0, ls1, ls2, ls3)
        gsems = (gs0, gs1)
        asems = (as0, as1)

        zvec = jnp.zeros((16,), jnp.float32)

        @pl.loop(0, ZROWS)
        def _zr(r):
            for j in range(HALF // 16):
                zero_v[r, pl.ds(j * 16, 16)] = zvec

        for i, (a_hbm, wgt_hbm) in enumerate(
                ((a0_hbm, w0_hbm), (a1_hbm, w1_hbm), (a2_hbm, w2_hbm))):
            rowoff = (2 * i) * N + c * N

            def fire_l(r, ordinal, _a=a_hbm, _w=wgt_hbm):
                base = (s + NS * ordinal) * CHUNK
                pltpu.async_copy(_a.at[1].at[pl.ds(base, CHUNK)], srcs[r], lsems[r])
                pltpu.async_copy(_a.at[0].at[pl.ds(base, CHUNK)], dsts[r], lsems[r])
                pltpu.async_copy(_w.at[pl.ds(base, CHUNK)], wvs[r], lsems[r])

            def wait_l(r, _a=a_hbm, _w=wgt_hbm):
                pltpu.make_async_copy(_a.at[1].at[pl.ds(0, CHUNK)], srcs[r], lsems[r]).wait()
                pltpu.make_async_copy(_a.at[0].at[pl.ds(0, CHUNK)], dsts[r], lsems[r]).wait()
                pltpu.make_async_copy(_w.at[pl.ds(0, CHUNK)], wvs[r], lsems[r]).wait()

            def adjust(r, _off=rowoff):
                offv = jnp.full((16,), _off, jnp.int32)
                for u in range(CHUNK // 16):
                    srcs[r][pl.ds(u * 16, 16)] = srcs[r][pl.ds(u * 16, 16)] + offv

            def fire_g(r, b):
                pltpu.async_copy(h_hbm.at[srcs[r]], rbfs[b], gsems[b])

            def wait_g(r, b):
                pltpu.make_async_copy(h_hbm.at[srcs[r]], rbfs[b], gsems[b]).wait()

            def scale(r, b):
                @plsc.parallel_loop(0, CHUNK, unroll=4)
                def _scale(e):
                    wb = plsc.load_gather(wvs[r], [jnp.full((16,), e, jnp.int32)])
                    for q in range(HALF // 32):
                        packed = rbfs[b][e, pl.ds(q * 16, 16)]
                        lo = plsc.bitcast(packed << 16, jnp.float32)
                        hi = plsc.bitcast(
                            packed & jnp.int32(-65536), jnp.float32)
                        rows[b][e, pl.ds(q * 32, 16)] = lo * wb
                        rows[b][e, pl.ds(q * 32 + 16, 16)] = hi * wb

            def fire_a(r, b):
                for u in range(CHUNK // 16):
                    dscs[b][pl.ds(u * 16, 16)] = dsts[r][pl.ds(u * 16, 16)]
                pltpu.async_copy(rows[b], acc_sh.at[dscs[b]], asems[b], add=True)

            def wait_a(b):
                pltpu.make_async_copy(rows[b], acc_sh.at[dscs[b]], asems[b]).wait()

            for r in range(4):
                fire_l(r, r)

            for q in range(ROWS_MAIN // ZROWS):
                pltpu.async_copy(
                    zero_v,
                    acc_sh.at[pl.ds(s * ROWS_MAIN + q * ZROWS, ZROWS)], zs)

            @pl.when(s == 0)
            def _ztail():
                pltpu.async_copy(zero_v.at[pl.ds(0, ROWS_TAIL)],
                                 acc_sh.at[pl.ds(NS * ROWS_MAIN, ROWS_TAIL)], zs)

            for q in range(ROWS_MAIN // ZROWS):
                pltpu.make_async_copy(
                    zero_v,
                    acc_sh.at[pl.ds(s * ROWS_MAIN + q * ZROWS, ZROWS)], zs).wait()

            @pl.when(s == 0)
            def _ztailw():
                pltpu.make_async_copy(
                    zero_v.at[pl.ds(0, ROWS_TAIL)],
                    acc_sh.at[pl.ds(NS * ROWS_MAIN, ROWS_TAIL)], zs).wait()

            plsc.subcore_barrier()

            @pl.loop(0, MAIN_ITERS)
            def _main(u):
                wait_l(0)
                adjust(0)

                @pl.when(u > 0)
                def _wa0():
                    wait_a(0)

                fire_g(0, 0)

                wait_l(1)
                adjust(1)

                @pl.when(u > 0)
                def _wa1():
                    wait_a(1)

                fire_g(1, 1)

                wait_g(0, 0)
                scale(0, 0)
                fire_a(0, 0)

                @pl.when(4 * u + 4 < ORDS)
                def _pf0():
                    fire_l(0, 4 * u + 4)

                wait_l(2)
                adjust(2)
                wait_a(0)
                fire_g(2, 0)

                wait_g(1, 1)
                scale(1, 1)
                fire_a(1, 1)

                @pl.when(4 * u + 5 < ORDS)
                def _pf1():
                    fire_l(1, 4 * u + 5)

                wait_l(3)
                adjust(3)
                wait_a(1)
                fire_g(3, 1)

                wait_g(2, 0)
                scale(2, 0)
                fire_a(2, 0)

                @pl.when(u + 1 < MAIN_ITERS)
                def _pf2():
                    fire_l(2, 4 * u + 6)

                wait_g(3, 1)
                scale(3, 1)
                fire_a(3, 1)

                @pl.when(u + 1 < MAIN_ITERS)
                def _pf3():
                    fire_l(3, 4 * u + 7)

            for j in range(EPI):
                wait_l(j)
                adjust(j)
                wait_a(j)
                fire_g(j, j)
            for j in range(EPI):
                wait_g(j, j)
                scale(j, j)
                fire_a(j, j)
            wait_a(0)
            wait_a(1)

            @pl.when(s < LEFT)
            def _left():
                base = (ORDS * NS + s) * CHUNK
                pltpu.async_copy(a_hbm.at[1].at[pl.ds(base, CHUNK)], src0, ls0)
                pltpu.async_copy(a_hbm.at[0].at[pl.ds(base, CHUNK)], dst0, ls0)
                pltpu.async_copy(wgt_hbm.at[pl.ds(base, CHUNK)], wv0, ls0).wait()
                pltpu.make_async_copy(a_hbm.at[1].at[pl.ds(0, CHUNK)], src0, ls0).wait()
                pltpu.make_async_copy(a_hbm.at[0].at[pl.ds(0, CHUNK)], dst0, ls0).wait()
                adjust(0)
                pltpu.async_copy(h_hbm.at[src0], rbf0, gs0).wait()
                scale(0, 0)
                pltpu.async_copy(rows0, acc_sh.at[dst0], as0, add=True).wait()

            plsc.subcore_barrier()
            pltpu.sync_copy(
                acc_sh.at[pl.ds(s * ROWS_MAIN, ROWS_MAIN)],
                out_hbm.at[pl.ds(rowoff + s * ROWS_MAIN, ROWS_MAIN)])

            @pl.when(s == 0)
            def _wtail():
                pltpu.sync_copy(
                    acc_sh.at[pl.ds(NS * ROWS_MAIN, ROWS_TAIL)],
                    out_hbm.at[pl.ds(rowoff + NS * ROWS_MAIN, ROWS_TAIL)])

    return spmm


_spmm = _make_spmm()


def _attn_body(m4_ref, b_ref, asl_ref, wlt_ref, bl_ref, esum_ref):
    @pl.when(pl.program_id(0) == 0)
    def _():
        esum_ref[...] = jnp.zeros_like(esum_ref)

    x = m4_ref[...]
    for i in range(3):
        m = jnp.concatenate([x[i, 0], x[i, 1]], axis=-1)
        m = m + b_ref[i]
        m = jnp.where(m >= 0, m, m * asl_ref[i])
        t = lax.dot_general(m, wlt_ref[...], (((1,), (0,)), ((), ())),
                            preferred_element_type=jnp.float32,
                            precision=lax.Precision.HIGHEST)
        t = jnp.tanh(t + bl_ref[...])
        esum_ref[i] = esum_ref[i] + jnp.sum(t, axis=0)


def _attn_sums(m4, bs, asl, wlt, bl_row):
    return pl.pallas_call(
        _attn_body,
        grid=(NB,),
        in_specs=[
            pl.BlockSpec((3, 2, BN, HALF), lambda b: (0, 0, b, 0)),
            pl.BlockSpec((3, H), lambda b: (0, 0)),
            pl.BlockSpec((3, 1), lambda b: (0, 0)),
            pl.BlockSpec((H, H), lambda b: (0, 0)),
            pl.BlockSpec((1, H), lambda b: (0, 0)),
        ],
        out_specs=pl.BlockSpec((3, H), lambda b: (0, 0)),
        out_shape=jax.ShapeDtypeStruct((3, H), jnp.float32),
    )(m4, bs, asl, wlt, bl_row)


def _z_body(m4_ref, esum_ref, b_ref, asl_ref, beta_ref, z_ref):
    es = esum_ref[...]
    logits = jnp.sum(es * beta_ref[...], axis=1) * (1.0 / N)
    lmax = jnp.max(logits)
    p = jnp.exp(logits - lmax)
    a = p / jnp.sum(p)

    x = m4_ref[...]
    z = jnp.zeros((x.shape[2], H), jnp.float32)
    for i in range(3):
        m = jnp.concatenate([x[i, 0], x[i, 1]], axis=-1)
        m = m + b_ref[i]
        m = jnp.where(m >= 0, m, m * asl_ref[i])
        z = z + m * a[i:i + 1]
    z_ref[...] = z


def _z_final(m4, esum, bs, asl, beta_row):
    return pl.pallas_call(
        _z_body,
        grid=(NB,),
        in_specs=[
            pl.BlockSpec((3, 2, BN, HALF), lambda b: (0, 0, b, 0)),
            pl.BlockSpec((3, H), lambda b: (0, 0)),
            pl.BlockSpec((3, H), lambda b: (0, 0)),
            pl.BlockSpec((3, 1), lambda b: (0, 0)),
            pl.BlockSpec((1, H), lambda b: (0, 0)),
        ],
        out_specs=pl.BlockSpec((BN, H), lambda b: (b, 0)),
        out_shape=jax.ShapeDtypeStruct((N, H), jnp.float32),
    )(m4, esum, bs, asl, beta_row)


def kernel(pro_feature, adj0_idx, adj0_w, adj1_idx, adj1_w, adj2_idx, adj2_w,
           W0, b0, a0, W1, b1, a1, W2, b2, a2, Wl, bl, beta):
    w6t = jnp.stack([W0, W1, W2]).reshape(3, 2, HALF, H)
    w6t = w6t.transpose(0, 1, 3, 2).reshape(6, H, HALF)
    perm = np.empty((HALF,), dtype=np.int32)
    for q in range(HALF // 32):
        for t in range(16):
            perm[16 * q + t] = 32 * q + t
            perm[64 + 16 * q + t] = 32 * q + 16 + t
    w6t = w6t[:, :, perm]

    h6 = _matmul_h(pro_feature, w6t)
    hr = h6.reshape(6 * N, HALF)
    lo16 = jax.lax.bitcast_convert_type(hr[:, :HALF // 2], jnp.uint16)
    hi16 = jax.lax.bitcast_convert_type(hr[:, HALF // 2:], jnp.uint16)
    hpack = (lo16.astype(jnp.int32)
             | (hi16.astype(jnp.int32) << 16))
    out6 = _spmm(hpack,
                 adj0_idx, adj0_w, adj1_idx, adj1_w, adj2_idx, adj2_w)
    m4 = out6.reshape(3, 2, N, HALF)

    bs = jnp.stack([b0, b1, b2])
    asl = jnp.stack([a0, a1, a2]).reshape(3, 1)
    esum = _attn_sums(m4, bs, asl, Wl.T, bl.reshape(1, H))
    return _z_final(m4, esum, bs, asl, beta.reshape(1, H))


# speedup vs baseline: 1.1213x; 1.1213x over previous
import dataclasses
import functools

import jax
import jax.numpy as jnp
from jax import lax
from jax.experimental import pallas as pl
from jax.experimental.pallas import tpu as pltpu
from jax.experimental.pallas import tpu_sc as plsc

N = 10000
E = 160000
H = 256
HALF = 128

NC = 2
NS = 16
CHUNK = 128
NCHUNK_TOT = E // CHUNK
ORDS = NCHUNK_TOT // NS
LEFT = NCHUNK_TOT - ORDS * NS
DEPTH = 3
MAIN_ITERS = ORDS // DEPTH
EPI = ORDS - MAIN_ITERS * DEPTH
ROWS_MAIN = 624
ROWS_TAIL = N - ROWS_MAIN * NS
ZSPLIT = (128, 128, 128, 128, 112)

BN = 2000
NB = N // BN


def _mm_body(x_ref, w_ref, o_ref):
    o_ref[0] = lax.dot_general(
        x_ref[...], w_ref[0],
        (((1,), (0,)), ((), ())),
        preferred_element_type=jnp.float32,
        precision=lax.Precision.HIGHEST)


def _matmul_h(x, w6t):
    return pl.pallas_call(
        _mm_body,
        grid=(NB, 6),
        in_specs=[
            pl.BlockSpec((BN, H), lambda b, g: (b, 0)),
            pl.BlockSpec((1, H, HALF), lambda b, g: (g, 0, 0)),
        ],
        out_specs=pl.BlockSpec((1, BN, HALF), lambda b, g: (g, b, 0)),
        out_shape=jax.ShapeDtypeStruct((6, N, HALF), jnp.float32),
    )(x, w6t)


def _make_spmm():
    mesh = plsc.VectorSubcoreMesh(core_axis_name="c", subcore_axis_name="s")
    cp = pltpu.CompilerParams()
    if "needs_layout_passes" in pltpu.CompilerParams.__dataclass_fields__:
        cp = dataclasses.replace(cp, needs_layout_passes=False)

    @functools.partial(
        pl.kernel,
        mesh=mesh,
        compiler_params=cp,
        out_type=jax.ShapeDtypeStruct((6 * N, HALF), jnp.float32),
        scratch_types=(
            [pltpu.VMEM((CHUNK,), jnp.int32) for _ in range(DEPTH)]
            + [pltpu.VMEM((CHUNK,), jnp.int32) for _ in range(DEPTH)]
            + [pltpu.VMEM((CHUNK,), jnp.float32) for _ in range(DEPTH)]
            + [pltpu.VMEM((CHUNK,), jnp.int32) for _ in range(DEPTH)]
            + [pltpu.VMEM((CHUNK, HALF), jnp.float32) for _ in range(DEPTH)]
            + [pltpu.VMEM_SHARED((N, HALF), jnp.float32)]
            + [pltpu.SemaphoreType.DMA for _ in range(3 * DEPTH + 1)]
        ),
    )
    def spmm(h_hbm, a0_hbm, w0_hbm, a1_hbm, w1_hbm, a2_hbm, w2_hbm, out_hbm,
             src0, src1, src2, dst0, dst1, dst2,
             wv0, wv1, wv2, dsc0, dsc1, dsc2,
             rows0, rows1, rows2, acc_sh,
             ls0, ls1, ls2, gs0, gs1, gs2, as0, as1, as2, zs):
        c = lax.axis_index("c")
        s = lax.axis_index("s")
        srcs = (src0, src1, src2)
        dsts = (dst0, dst1, dst2)
        wvs = (wv0, wv1, wv2)
        dscs = (dsc0, dsc1, dsc2)
        rows = (rows0, rows1, rows2)
        lsems = (ls0, ls1, ls2)
        gsems = (gs0, gs1, gs2)
        asems = (as0, as1, as2)

        zvec = jnp.zeros((16,), jnp.float32)

        for i, (a_hbm, wgt_hbm) in enumerate(
                ((a0_hbm, w0_hbm), (a1_hbm, w1_hbm), (a2_hbm, w2_hbm))):
            rowoff = (2 * i) * N + c * N

            def fire_l(r, ordinal, _a=a_hbm, _w=wgt_hbm):
                base = (s + NS * ordinal) * CHUNK
                pltpu.async_copy(_a.at[1].at[pl.ds(base, CHUNK)], srcs[r], lsems[r])
                pltpu.async_copy(_a.at[0].at[pl.ds(base, CHUNK)], dsts[r], lsems[r])
                pltpu.async_copy(_w.at[pl.ds(base, CHUNK)], wvs[r], lsems[r])

            def wait_l(r, _a=a_hbm, _w=wgt_hbm):
                pltpu.make_async_copy(_a.at[1].at[pl.ds(0, CHUNK)], srcs[r], lsems[r]).wait()
                pltpu.make_async_copy(_a.at[0].at[pl.ds(0, CHUNK)], dsts[r], lsems[r]).wait()
                pltpu.make_async_copy(_w.at[pl.ds(0, CHUNK)], wvs[r], lsems[r]).wait()

            def adjust(r, _off=rowoff):
                offv = jnp.full((16,), _off, jnp.int32)
                for u in range(CHUNK // 16):
                    srcs[r][pl.ds(u * 16, 16)] = srcs[r][pl.ds(u * 16, 16)] + offv

            def fire_g(r):
                pltpu.async_copy(h_hbm.at[srcs[r]], rows[r], gsems[r])

            def wait_g(r):
                pltpu.make_async_copy(h_hbm.at[srcs[r]], rows[r], gsems[r]).wait()

            def scale(r):
                @plsc.parallel_loop(0, CHUNK, unroll=4)
                def _scale(e):
                    wb = plsc.load_gather(wvs[r], [jnp.full((16,), e, jnp.int32)])
                    for j in range(HALF // 16):
                        rows[r][e, pl.ds(j * 16, 16)] = (
                            rows[r][e, pl.ds(j * 16, 16)] * wb)

            def fire_a(r):
                for u in range(CHUNK // 16):
                    dscs[r][pl.ds(u * 16, 16)] = dsts[r][pl.ds(u * 16, 16)]
                pltpu.async_copy(rows[r], acc_sh.at[dscs[r]], asems[r], add=True)

            def wait_a(r):
                pltpu.make_async_copy(rows[r], acc_sh.at[dscs[r]], asems[r]).wait()

            for r in range(DEPTH):
                fire_l(r, r)

            @pl.loop(0, CHUNK)
            def _zr(r):
                for j in range(HALF // 16):
                    rows0[r, pl.ds(j * 16, 16)] = zvec

            zoff = 0
            for zn in ZSPLIT:
                pltpu.async_copy(
                    rows0.at[pl.ds(0, zn)],
                    acc_sh.at[pl.ds(s * ROWS_MAIN + zoff, zn)], zs)
                zoff += zn

            @pl.when(s == 0)
            def _ztail():
                pltpu.async_copy(rows0.at[pl.ds(0, ROWS_TAIL)],
                                 acc_sh.at[pl.ds(NS * ROWS_MAIN, ROWS_TAIL)], zs)

            zoff = 0
            for zn in ZSPLIT:
                pltpu.make_async_copy(
                    rows0.at[pl.ds(0, zn)],
                    acc_sh.at[pl.ds(s * ROWS_MAIN + zoff, zn)], zs).wait()
                zoff += zn

            @pl.when(s == 0)
            def _ztailw():
                pltpu.make_async_copy(
                    rows0.at[pl.ds(0, ROWS_TAIL)],
                    acc_sh.at[pl.ds(NS * ROWS_MAIN, ROWS_TAIL)], zs).wait()

            plsc.subcore_barrier()

            @pl.loop(0, MAIN_ITERS)
            def _main(u):
                for j in range(DEPTH):
                    wait_l(j)
                    adjust(j)

                    @pl.when(u > 0)
                    def _wa():
                        wait_a(j)

                    fire_g(j)
                for j in range(DEPTH):
                    wait_g(j)
                    scale(j)
                    fire_a(j)

                    @pl.when(u + 1 < MAIN_ITERS)
                    def _pf():
                        fire_l(j, DEPTH * u + DEPTH + j)

            for j in range(DEPTH):
                wait_a(j)

            @pl.when(s < LEFT)
            def _left():
                base = (ORDS * NS + s) * CHUNK
                pltpu.async_copy(a_hbm.at[1].at[pl.ds(base, CHUNK)], src0, ls0)
                pltpu.async_copy(a_hbm.at[0].at[pl.ds(base, CHUNK)], dst0, ls0)
                pltpu.async_copy(wgt_hbm.at[pl.ds(base, CHUNK)], wv0, ls0).wait()
                pltpu.make_async_copy(a_hbm.at[1].at[pl.ds(0, CHUNK)], src0, ls0).wait()
                pltpu.make_async_copy(a_hbm.at[0].at[pl.ds(0, CHUNK)], dst0, ls0).wait()
                adjust(0)
                pltpu.async_copy(h_hbm.at[src0], rows0, gs0).wait()
                scale(0)
                pltpu.async_copy(rows0, acc_sh.at[dst0], as0, add=True).wait()

            plsc.subcore_barrier()
            pltpu.sync_copy(
                acc_sh.at[pl.ds(s * ROWS_MAIN, ROWS_MAIN)],
                out_hbm.at[pl.ds(rowoff + s * ROWS_MAIN, ROWS_MAIN)])

            @pl.when(s == 0)
            def _wtail():
                pltpu.sync_copy(
                    acc_sh.at[pl.ds(NS * ROWS_MAIN, ROWS_TAIL)],
                    out_hbm.at[pl.ds(rowoff + NS * ROWS_MAIN, ROWS_TAIL)])

    return spmm


_spmm = _make_spmm()


def _attn_body(m4_ref, b_ref, asl_ref, wlt_ref, bl_ref, esum_ref):
    @pl.when(pl.program_id(0) == 0)
    def _():
        esum_ref[...] = jnp.zeros_like(esum_ref)

    x = m4_ref[...]
    for i in range(3):
        m = jnp.concatenate([x[i, 0], x[i, 1]], axis=-1)
        m = m + b_ref[i]
        m = jnp.where(m >= 0, m, m * asl_ref[i])
        t = lax.dot_general(m, wlt_ref[...], (((1,), (0,)), ((), ())),
                            preferred_element_type=jnp.float32,
                            precision=lax.Precision.HIGHEST)
        t = jnp.tanh(t + bl_ref[...])
        esum_ref[i] = esum_ref[i] + jnp.sum(t, axis=0)


def _attn_sums(m4, bs, asl, wlt, bl_row):
    return pl.pallas_call(
        _attn_body,
        grid=(NB,),
        in_specs=[
            pl.BlockSpec((3, 2, BN, HALF), lambda b: (0, 0, b, 0)),
            pl.BlockSpec((3, H), lambda b: (0, 0)),
            pl.BlockSpec((3, 1), lambda b: (0, 0)),
            pl.BlockSpec((H, H), lambda b: (0, 0)),
            pl.BlockSpec((1, H), lambda b: (0, 0)),
        ],
        out_specs=pl.BlockSpec((3, H), lambda b: (0, 0)),
        out_shape=jax.ShapeDtypeStruct((3, H), jnp.float32),
    )(m4, bs, asl, wlt, bl_row)


def _z_body(m4_ref, esum_ref, b_ref, asl_ref, beta_ref, z_ref):
    es = esum_ref[...]
    logits = jnp.sum(es * beta_ref[...], axis=1) * (1.0 / N)
    lmax = jnp.max(logits)
    p = jnp.exp(logits - lmax)
    a = p / jnp.sum(p)

    x = m4_ref[...]
    z = jnp.zeros((x.shape[2], H), jnp.float32)
    for i in range(3):
        m = jnp.concatenate([x[i, 0], x[i, 1]], axis=-1)
        m = m + b_ref[i]
        m = jnp.where(m >= 0, m, m * asl_ref[i])
        z = z + m * a[i:i + 1]
    z_ref[...] = z


def _z_final(m4, esum, bs, asl, beta_row):
    return pl.pallas_call(
        _z_body,
        grid=(NB,),
        in_specs=[
            pl.BlockSpec((3, 2, BN, HALF), lambda b: (0, 0, b, 0)),
            pl.BlockSpec((3, H), lambda b: (0, 0)),
            pl.BlockSpec((3, H), lambda b: (0, 0)),
            pl.BlockSpec((3, 1), lambda b: (0, 0)),
            pl.BlockSpec((1, H), lambda b: (0, 0)),
        ],
        out_specs=pl.BlockSpec((BN, H), lambda b: (b, 0)),
        out_shape=jax.ShapeDtypeStruct((N, H), jnp.float32),
    )(m4, esum, bs, asl, beta_row)


def kernel(pro_feature, adj0_idx, adj0_w, adj1_idx, adj1_w, adj2_idx, adj2_w,
           W0, b0, a0, W1, b1, a1, W2, b2, a2, Wl, bl, beta):
    w6t = jnp.stack([W0, W1, W2]).reshape(3, 2, HALF, H)
    w6t = w6t.transpose(0, 1, 3, 2).reshape(6, H, HALF)

    h6 = _matmul_h(pro_feature, w6t)
    out6 = _spmm(h6.reshape(6 * N, HALF),
                 adj0_idx, adj0_w, adj1_idx, adj1_w, adj2_idx, adj2_w)
    m4 = out6.reshape(3, 2, N, HALF)

    bs = jnp.stack([b0, b1, b2])
    asl = jnp.stack([a0, a1, a2]).reshape(3, 1)
    esum = _attn_sums(m4, bs, asl, Wl.T, bl.reshape(1, H))
    return _z_final(m4, esum, bs, asl, beta.reshape(1, H))


# speedup vs baseline: 1.1426x; 1.0190x over previous
import dataclasses
import functools

import jax
import jax.numpy as jnp
from jax import lax
from jax.experimental import pallas as pl
from jax.experimental.pallas import tpu as pltpu
from jax.experimental.pallas import tpu_sc as plsc

N = 10000
E = 160000
H = 256
HALF = 128

NC = 2
NS = 16
CHUNK = 128
NCHUNK_TOT = E // CHUNK
ORDS = NCHUNK_TOT // NS
LEFT = NCHUNK_TOT - ORDS * NS
DEPTH = 3
MAIN_ITERS = ORDS // DEPTH
EPI = ORDS - MAIN_ITERS * DEPTH
ROWS_MAIN = 624
ROWS_TAIL = N - ROWS_MAIN * NS
ZSPLIT = (128, 128, 128, 128, 112)

BN = 2000
NB = N // BN


def _mm_body(x_ref, w_ref, o_ref):
    o_ref[0] = lax.dot_general(
        x_ref[...], w_ref[0],
        (((1,), (0,)), ((), ())),
        preferred_element_type=jnp.float32,
        precision=lax.Precision.HIGHEST)


def _matmul_h(x, w6t):
    return pl.pallas_call(
        _mm_body,
        grid=(NB, 6),
        in_specs=[
            pl.BlockSpec((BN, H), lambda b, g: (b, 0)),
            pl.BlockSpec((1, H, HALF), lambda b, g: (g, 0, 0)),
        ],
        out_specs=pl.BlockSpec((1, BN, HALF), lambda b, g: (g, b, 0)),
        out_shape=jax.ShapeDtypeStruct((6, N, HALF), jnp.float32),
    )(x, w6t)


def _make_spmm(i):
    mesh = plsc.VectorSubcoreMesh(core_axis_name="c", subcore_axis_name="s")
    cp = pltpu.CompilerParams()
    if "needs_layout_passes" in pltpu.CompilerParams.__dataclass_fields__:
        cp = dataclasses.replace(cp, needs_layout_passes=False)

    @functools.partial(
        pl.kernel,
        mesh=mesh,
        compiler_params=cp,
        out_type=jax.ShapeDtypeStruct((2 * N, HALF), jnp.float32),
        scratch_types=(
            [pltpu.VMEM((CHUNK,), jnp.int32) for _ in range(DEPTH)]
            + [pltpu.VMEM((CHUNK,), jnp.int32) for _ in range(DEPTH)]
            + [pltpu.VMEM((CHUNK,), jnp.float32) for _ in range(DEPTH)]
            + [pltpu.VMEM((CHUNK,), jnp.int32) for _ in range(DEPTH)]
            + [pltpu.VMEM((CHUNK, HALF), jnp.float32) for _ in range(DEPTH)]
            + [pltpu.VMEM_SHARED((N, HALF), jnp.float32)]
            + [pltpu.SemaphoreType.DMA for _ in range(3 * DEPTH + 1)]
        ),
    )
    def spmm(h_hbm, a_hbm, wgt_hbm, out_hbm,
             src0, src1, src2, dst0, dst1, dst2,
             wv0, wv1, wv2, dsc0, dsc1, dsc2,
             rows0, rows1, rows2, acc_sh,
             ls0, ls1, ls2, gs0, gs1, gs2, as0, as1, as2, zs):
        c = lax.axis_index("c")
        s = lax.axis_index("s")
        srcs = (src0, src1, src2)
        dsts = (dst0, dst1, dst2)
        wvs = (wv0, wv1, wv2)
        dscs = (dsc0, dsc1, dsc2)
        rows = (rows0, rows1, rows2)
        lsems = (ls0, ls1, ls2)
        gsems = (gs0, gs1, gs2)
        asems = (as0, as1, as2)

        zvec = jnp.zeros((16,), jnp.float32)
        rowoff = (2 * i) * N + c * N
        outoff = c * N

        def fire_l(r, ordinal):
            base = (s + NS * ordinal) * CHUNK
            pltpu.async_copy(a_hbm.at[1].at[pl.ds(base, CHUNK)], srcs[r], lsems[r])
            pltpu.async_copy(a_hbm.at[0].at[pl.ds(base, CHUNK)], dsts[r], lsems[r])
            pltpu.async_copy(wgt_hbm.at[pl.ds(base, CHUNK)], wvs[r], lsems[r])

        def wait_l(r):
            pltpu.make_async_copy(a_hbm.at[1].at[pl.ds(0, CHUNK)], srcs[r], lsems[r]).wait()
            pltpu.make_async_copy(a_hbm.at[0].at[pl.ds(0, CHUNK)], dsts[r], lsems[r]).wait()
            pltpu.make_async_copy(wgt_hbm.at[pl.ds(0, CHUNK)], wvs[r], lsems[r]).wait()

        def adjust(r):
            offv = jnp.full((16,), rowoff, jnp.int32)
            for u in range(CHUNK // 16):
                srcs[r][pl.ds(u * 16, 16)] = srcs[r][pl.ds(u * 16, 16)] + offv

        def fire_g(r):
            pltpu.async_copy(h_hbm.at[srcs[r]], rows[r], gsems[r])

        def wait_g(r):
            pltpu.make_async_copy(h_hbm.at[srcs[r]], rows[r], gsems[r]).wait()

        def scale(r):
            @plsc.parallel_loop(0, CHUNK, unroll=4)
            def _scale(e):
                wb = plsc.load_gather(wvs[r], [jnp.full((16,), e, jnp.int32)])
                for j in range(HALF // 16):
                    rows[r][e, pl.ds(j * 16, 16)] = (
                        rows[r][e, pl.ds(j * 16, 16)] * wb)

        def fire_a(r):
            for u in range(CHUNK // 16):
                dscs[r][pl.ds(u * 16, 16)] = dsts[r][pl.ds(u * 16, 16)]
            pltpu.async_copy(rows[r], acc_sh.at[dscs[r]], asems[r], add=True)

        def wait_a(r):
            pltpu.make_async_copy(rows[r], acc_sh.at[dscs[r]], asems[r]).wait()

        for r in range(DEPTH):
            fire_l(r, r)

        @pl.loop(0, CHUNK)
        def _zr(r):
            for j in range(HALF // 16):
                rows0[r, pl.ds(j * 16, 16)] = zvec

        zoff = 0
        for zn in ZSPLIT:
            pltpu.async_copy(
                rows0.at[pl.ds(0, zn)],
                acc_sh.at[pl.ds(s * ROWS_MAIN + zoff, zn)], zs)
            zoff += zn

        @pl.when(s == 0)
        def _ztail():
            pltpu.async_copy(rows0.at[pl.ds(0, ROWS_TAIL)],
                             acc_sh.at[pl.ds(NS * ROWS_MAIN, ROWS_TAIL)], zs)

        zoff = 0
        for zn in ZSPLIT:
            pltpu.make_async_copy(
                rows0.at[pl.ds(0, zn)],
                acc_sh.at[pl.ds(s * ROWS_MAIN + zoff, zn)], zs).wait()
            zoff += zn

        @pl.when(s == 0)
        def _ztailw():
            pltpu.make_async_copy(
                rows0.at[pl.ds(0, ROWS_TAIL)],
                acc_sh.at[pl.ds(NS * ROWS_MAIN, ROWS_TAIL)], zs).wait()

        plsc.subcore_barrier()

        @pl.loop(0, MAIN_ITERS)
        def _main(u):
            for j in range(DEPTH):
                wait_l(j)
                adjust(j)

                @pl.when(u > 0)
                def _wa():
                    wait_a(j)

                fire_g(j)
            for j in range(DEPTH):
                wait_g(j)
                scale(j)
                fire_a(j)

                @pl.when(u + 1 < MAIN_ITERS)
                def _pf():
                    fire_l(j, DEPTH * u + DEPTH + j)

        for j in range(DEPTH):
            wait_a(j)

        @pl.when(s < LEFT)
        def _left():
            base = (ORDS * NS + s) * CHUNK
            pltpu.async_copy(a_hbm.at[1].at[pl.ds(base, CHUNK)], src0, ls0)
            pltpu.async_copy(a_hbm.at[0].at[pl.ds(base, CHUNK)], dst0, ls0)
            pltpu.async_copy(wgt_hbm.at[pl.ds(base, CHUNK)], wv0, ls0).wait()
            pltpu.make_async_copy(a_hbm.at[1].at[pl.ds(0, CHUNK)], src0, ls0).wait()
            pltpu.make_async_copy(a_hbm.at[0].at[pl.ds(0, CHUNK)], dst0, ls0).wait()
            adjust(0)
            pltpu.async_copy(h_hbm.at[src0], rows0, gs0).wait()
            scale(0)
            pltpu.async_copy(rows0, acc_sh.at[dst0], as0, add=True).wait()

        plsc.subcore_barrier()
        pltpu.sync_copy(
            acc_sh.at[pl.ds(s * ROWS_MAIN, ROWS_MAIN)],
            out_hbm.at[pl.ds(outoff + s * ROWS_MAIN, ROWS_MAIN)])

        @pl.when(s == 0)
        def _wtail():
            pltpu.sync_copy(
                acc_sh.at[pl.ds(NS * ROWS_MAIN, ROWS_TAIL)],
                out_hbm.at[pl.ds(outoff + NS * ROWS_MAIN, ROWS_TAIL)])

    return spmm


_spmm0 = _make_spmm(0)
_spmm1 = _make_spmm(1)
_spmm2 = _make_spmm(2)


def _attn_body(m2_ref, b_ref, asl_ref, wlt_ref, bl_ref, esum_ref):
    @pl.when(pl.program_id(0) == 0)
    def _():
        esum_ref[...] = jnp.zeros_like(esum_ref)

    x = m2_ref[...]
    m = jnp.concatenate([x[0], x[1]], axis=-1)
    m = m + b_ref[0]
    m = jnp.where(m >= 0, m, m * asl_ref[0])
    t = lax.dot_general(m, wlt_ref[...], (((1,), (0,)), ((), ())),
                        preferred_element_type=jnp.float32,
                        precision=lax.Precision.HIGHEST)
    t = jnp.tanh(t + bl_ref[...])
    esum_ref[0] = esum_ref[0] + jnp.sum(t, axis=0)


def _attn_sums(m2, b_row, asl1, wlt, bl_row):
    return pl.pallas_call(
        _attn_body,
        grid=(NB,),
        in_specs=[
            pl.BlockSpec((2, BN, HALF), lambda b: (0, b, 0)),
            pl.BlockSpec((1, H), lambda b: (0, 0)),
            pl.BlockSpec((1, 1), lambda b: (0, 0)),
            pl.BlockSpec((H, H), lambda b: (0, 0)),
            pl.BlockSpec((1, H), lambda b: (0, 0)),
        ],
        out_specs=pl.BlockSpec((1, H), lambda b: (0, 0)),
        out_shape=jax.ShapeDtypeStruct((1, H), jnp.float32),
    )(m2, b_row, asl1, wlt, bl_row)


def _z_body(m0_ref, m1_ref, m2_ref, esum_ref, b_ref, asl_ref, beta_ref, z_ref):
    es = esum_ref[...]
    logits = jnp.sum(es * beta_ref[...], axis=1) * (1.0 / N)
    lmax = jnp.max(logits)
    p = jnp.exp(logits - lmax)
    a = p / jnp.sum(p)

    z = jnp.zeros((BN, H), jnp.float32)
    for i, mref in enumerate((m0_ref, m1_ref, m2_ref)):
        x = mref[...]
        m = jnp.concatenate([x[0], x[1]], axis=-1)
        m = m + b_ref[i]
        m = jnp.where(m >= 0, m, m * asl_ref[i])
        z = z + m * a[i:i + 1]
    z_ref[...] = z


def _z_final(m0, m1, m2, esum, bs, asl, beta_row):
    mspec = pl.BlockSpec((2, BN, HALF), lambda b: (0, b, 0))
    return pl.pallas_call(
        _z_body,
        grid=(NB,),
        in_specs=[
            mspec, mspec, mspec,
            pl.BlockSpec((3, H), lambda b: (0, 0)),
            pl.BlockSpec((3, H), lambda b: (0, 0)),
            pl.BlockSpec((3, 1), lambda b: (0, 0)),
            pl.BlockSpec((1, H), lambda b: (0, 0)),
        ],
        out_specs=pl.BlockSpec((BN, H), lambda b: (b, 0)),
        out_shape=jax.ShapeDtypeStruct((N, H), jnp.float32),
    )(m0, m1, m2, esum, bs, asl, beta_row)


def kernel(pro_feature, adj0_idx, adj0_w, adj1_idx, adj1_w, adj2_idx, adj2_w,
           W0, b0, a0, W1, b1, a1, W2, b2, a2, Wl, bl, beta):
    w6t = jnp.stack([W0, W1, W2]).reshape(3, 2, HALF, H)
    w6t = w6t.transpose(0, 1, 3, 2).reshape(6, H, HALF)

    h6 = _matmul_h(pro_feature, w6t).reshape(6 * N, HALF)
    wlt = Wl.T
    bl_row = bl.reshape(1, H)
    bs = jnp.stack([b0, b1, b2])
    asl = jnp.stack([a0, a1, a2]).reshape(3, 1)

    ms, es = [], []
    for i, (spmm_i, adj, w) in enumerate((
            (_spmm0, adj0_idx, adj0_w),
            (_spmm1, adj1_idx, adj1_w),
            (_spmm2, adj2_idx, adj2_w))):
        m2 = spmm_i(h6, adj, w).reshape(2, N, HALF)
        ms.append(m2)
        es.append(_attn_sums(m2, bs[i:i + 1], asl[i:i + 1], wlt, bl_row))

    esum = jnp.concatenate(es, axis=0)
    return _z_final(ms[0], ms[1], ms[2], esum, bs, asl, beta.reshape(1, H))
